# baseline (speedup 1.0000x reference)
import functools

import jax
import jax.numpy as jnp
from jax import lax
from jax.experimental import pallas as pl
from jax.experimental.pallas import tpu as pltpu
from jax.experimental.pallas import tpu_sc as plsc

NBITS = 100
DIM = 64
G = 6
NG = 17
TROWS = NG * 64 - 48
NC = 2
NS = 16
NW = NC * NS
LANES = 16
LASTROW = (NG - 1) * 64 * DIM


def _sc_embed(w_flat, msg_flat, n_batch):
    bpw = n_batch // NW
    qch = 8
    qrows = bpw // qch
    nbtq = qrows // LANES

    mesh = plsc.VectorSubcoreMesh(core_axis_name="c", subcore_axis_name="s")

    @functools.partial(
        pl.kernel,
        out_type=jax.ShapeDtypeStruct((n_batch * DIM,), jnp.float32),
        mesh=mesh,
        compiler_params=pltpu.CompilerParams(needs_layout_passes=False),
        scratch_types=[
            pltpu.VMEM((TROWS * DIM,), jnp.float32),
            pltpu.VMEM((NBITS * 2 * DIM,), jnp.float32),
            pltpu.VMEM((qrows * NBITS,), jnp.int32),
            pltpu.VMEM((qrows * NBITS,), jnp.int32),
            pltpu.VMEM((bpw * DIM,), jnp.float32),
            pltpu.VMEM((NG * LANES,), jnp.int32),
            pltpu.SemaphoreType.DMA,
            pltpu.SemaphoreType.DMA,
            pltpu.SemaphoreType.DMA,
            pltpu.SemaphoreType.DMA,
        ],
    )
    def sc_kernel(w_hbm, msg_hbm, out_hbm, t_v, w_v, m0_v, m1_v, out_v, mb_v,
                  sem_w, sem_m0, sem_m1, sem_out):
        cid = lax.axis_index("c")
        sid = lax.axis_index("s")
        wid = sid * NC + cid
        row0 = wid * bpw

        bufs = [m0_v, m1_v]
        sems = [sem_m0, sem_m1]

        wcp = pltpu.make_async_copy(w_hbm, w_v, sem_w)
        wcp.start()

        def msg_cp(q):
            return pltpu.make_async_copy(
                msg_hbm.at[pl.ds((row0 + q * qrows) * NBITS, qrows * NBITS)],
                bufs[q % 2], sems[q % 2])

        descs = {q: msg_cp(q) for q in range(2)}
        descs[0].start()
        descs[1].start()
        wcp.wait()

        zero = jnp.zeros((LANES,), jnp.float32)
        for s in range(DIM // LANES):
            base = lax.fori_loop(
                0, NBITS,
                lambda j, a, s=s: a + w_v[pl.ds(j * 2 * DIM + s * LANES, LANES)],
                zero)
            t_v[pl.ds(LASTROW + s * LANES, LANES)] = base

            def zrow(g, _, s=s):
                t_v[pl.ds(g * (64 * DIM) + s * LANES, LANES)] = zero
                return 0
            lax.fori_loop(0, NG - 1, zrow, 0)

        for k in range(G):
            ghi = NG if k < 4 else NG - 1

            def gstep(g, _, k=k):
                woff = (g * G + k) * 2 * DIM
                dsl = [w_v[pl.ds(woff + DIM + s * LANES, LANES)]
                       - w_v[pl.ds(woff + s * LANES, LANES)]
                       for s in range(DIM // LANES)]
                rb = g * (64 * DIM)

                def mstep(m, _):
                    src = rb + (m - (1 << k)) * DIM
                    dst = rb + m * DIM
                    for s in range(DIM // LANES):
                        t_v[pl.ds(dst + s * LANES, LANES)] = (
                            t_v[pl.ds(src + s * LANES, LANES)] + dsl[s])
                    return 0

                lax.fori_loop(1 << k, 2 << k, mstep, 0)
                return 0

            lax.fori_loop(0, ghi, gstep, 0)

        li = lax.iota(jnp.int32, LANES)

        def pair(p, _):
            for par in range(2):
                qd = 2 * p + par
                descs[par].wait()

                @pl.when(qd + 2 < qch)
                def _start_next(qd=qd, par=par):
                    pltpu.make_async_copy(
                        msg_hbm.at[pl.ds((row0 + (qd + 2) * qrows) * NBITS,
                                         qrows * NBITS)],
                        bufs[par], sems[par]).start()

                msg_v = bufs[par]

                def btile(bt, _, qd=qd, msg_v=msg_v):
                    ibase = (bt * LANES + li) * NBITS
                    obase = ((qd * qrows + bt * LANES) + li) * DIM
                    for g in range(NG):
                        nb = G if g < NG - 1 else NBITS - G * (NG - 1)
                        m = plsc.load_gather(msg_v, [ibase + G * g])
                        for i in range(1, nb):
                            bit = plsc.load_gather(msg_v, [ibase + (G * g + i)])
                            m = m + (bit << i)
                        mb_v[pl.ds(g * LANES, LANES)] = m

                    def cchunk(cc, _):
                        oadd = obase + cc * LANES
                        for g0, g1, first in ((0, 9, True), (9, NG, False)):
                            rbs = [
                                g * (64 * DIM) + cc * LANES
                                + mb_v[pl.ds(g * LANES, LANES)] * DIM
                                for g in range(g0, g1)
                            ]
                            for k in range(LANES):
                                sk = (li + k) & (LANES - 1)
                                acc = plsc.load_gather(t_v, [rbs[0] + sk])
                                for rb in rbs[1:]:
                                    acc = acc + plsc.load_gather(t_v, [rb + sk])
                                if first:
                                    plsc.store_scatter(out_v, [oadd + sk], acc)
                                else:
                                    plsc.addupdate_scatter(
                                        out_v, [oadd + sk], acc)
                        return 0

                    lax.fori_loop(0, DIM // LANES, cchunk, 0)
                    return 0

                lax.fori_loop(0, nbtq, btile, 0)
            return 0

        lax.fori_loop(0, qch // 2, pair, 0)

        ocp = pltpu.make_async_copy(
            out_v, out_hbm.at[pl.ds(row0 * DIM, bpw * DIM)], sem_out)
        ocp.start()
        ocp.wait()

    return sc_kernel(w_flat, msg_flat)


def kernel(msg, emb_weight):
    n_batch, n_bits = msg.shape
    out = _sc_embed(emb_weight.reshape(-1), msg.reshape(-1), n_batch)
    return out.reshape(n_batch, DIM)


# speedup vs baseline: 1.0009x; 1.0009x over previous
import functools

import jax
import jax.numpy as jnp
from jax import lax
from jax.experimental import pallas as pl
from jax.experimental.pallas import tpu as pltpu
from jax.experimental.pallas import tpu_sc as plsc

NBITS = 100
DIM = 64
G = 6
NG = 17
TROWS = NG * 64 - 48
NC = 2
NS = 16
NW = NC * NS
LANES = 16
LASTROW = (NG - 1) * 64 * DIM


def _sc_embed(w_flat, msg_flat, n_batch):
    bpw = n_batch // NW
    qch = 8
    qrows = bpw // qch
    nbtq = qrows // LANES

    mesh = plsc.VectorSubcoreMesh(core_axis_name="c", subcore_axis_name="s")

    @functools.partial(
        pl.kernel,
        out_type=jax.ShapeDtypeStruct((n_batch * DIM,), jnp.float32),
        mesh=mesh,
        compiler_params=pltpu.CompilerParams(needs_layout_passes=False),
        scratch_types=[
            pltpu.VMEM((TROWS * DIM,), jnp.float32),
            pltpu.VMEM((NBITS * 2 * DIM,), jnp.float32),
            pltpu.VMEM((qrows * NBITS,), jnp.int32),
            pltpu.VMEM((qrows * NBITS,), jnp.int32),
            pltpu.VMEM((bpw * DIM,), jnp.float32),
            pltpu.VMEM((NG * LANES,), jnp.int32),
            pltpu.SemaphoreType.DMA,
            pltpu.SemaphoreType.DMA,
            pltpu.SemaphoreType.DMA,
            pltpu.SemaphoreType.DMA,
        ],
    )
    def sc_kernel(w_hbm, msg_hbm, out_hbm, t_v, w_v, m0_v, m1_v, out_v, mb_v,
                  sem_w, sem_m0, sem_m1, sem_out):
        cid = lax.axis_index("c")
        sid = lax.axis_index("s")
        wid = sid * NC + cid
        row0 = wid * bpw

        bufs = [m0_v, m1_v]
        sems = [sem_m0, sem_m1]

        wcp = pltpu.make_async_copy(w_hbm, w_v, sem_w)
        wcp.start()

        def msg_cp(q):
            return pltpu.make_async_copy(
                msg_hbm.at[pl.ds((row0 + q * qrows) * NBITS, qrows * NBITS)],
                bufs[q % 2], sems[q % 2])

        descs = {q: msg_cp(q) for q in range(2)}
        descs[0].start()
        descs[1].start()
        wcp.wait()

        zero = jnp.zeros((LANES,), jnp.float32)
        for s in range(DIM // LANES):
            base = lax.fori_loop(
                0, NBITS,
                lambda j, a, s=s: a + w_v[pl.ds(j * 2 * DIM + s * LANES, LANES)],
                zero)
            t_v[pl.ds(LASTROW + s * LANES, LANES)] = base

            def zrow(g, _, s=s):
                t_v[pl.ds(g * (64 * DIM) + s * LANES, LANES)] = zero
                return 0
            lax.fori_loop(0, NG - 1, zrow, 0)

        for k in range(G):
            ghi = NG if k < 4 else NG - 1

            def gstep(g, _, k=k):
                woff = (g * G + k) * 2 * DIM
                dsl = [w_v[pl.ds(woff + DIM + s * LANES, LANES)]
                       - w_v[pl.ds(woff + s * LANES, LANES)]
                       for s in range(DIM // LANES)]
                rb = g * (64 * DIM)

                def mstep(m, _):
                    src = rb + (m - (1 << k)) * DIM
                    dst = rb + m * DIM
                    for s in range(DIM // LANES):
                        t_v[pl.ds(dst + s * LANES, LANES)] = (
                            t_v[pl.ds(src + s * LANES, LANES)] + dsl[s])
                    return 0

                lax.fori_loop(1 << k, 2 << k, mstep, 0)
                return 0

            lax.fori_loop(0, ghi, gstep, 0)

        li = lax.iota(jnp.int32, LANES)

        def pair(p, _):
            for par in range(2):
                qd = 2 * p + par
                descs[par].wait()
                msg_v = bufs[par]

                def btile(bt, _, qd=qd, msg_v=msg_v):
                    ibase = (bt * LANES + li) * NBITS
                    obase = ((qd * qrows + bt * LANES) + li) * DIM
                    for g in range(NG):
                        nb = G if g < NG - 1 else NBITS - G * (NG - 1)
                        m = plsc.load_gather(msg_v, [ibase + G * g])
                        for i in range(1, nb):
                            bit = plsc.load_gather(msg_v, [ibase + (G * g + i)])
                            m = m + (bit << i)
                        mb_v[pl.ds(g * LANES, LANES)] = m

                    def cchunk(cc, _):
                        oadd = obase + cc * LANES
                        for g0, g1, first in ((0, 9, True), (9, NG, False)):
                            rbs = [
                                g * (64 * DIM) + cc * LANES
                                + mb_v[pl.ds(g * LANES, LANES)] * DIM
                                for g in range(g0, g1)
                            ]
                            for k in range(LANES):
                                sk = (li + k) & (LANES - 1)
                                acc = plsc.load_gather(t_v, [rbs[0] + sk])
                                for rb in rbs[1:]:
                                    acc = acc + plsc.load_gather(t_v, [rb + sk])
                                if first:
                                    plsc.store_scatter(out_v, [oadd + sk], acc)
                                else:
                                    plsc.addupdate_scatter(
                                        out_v, [oadd + sk], acc)
                        return 0

                    lax.fori_loop(0, DIM // LANES, cchunk, 0)
                    return 0

                lax.fori_loop(0, nbtq, btile, 0)

                @pl.when(qd + 2 < qch)
                def _start_next(qd=qd, par=par):
                    pltpu.make_async_copy(
                        msg_hbm.at[pl.ds((row0 + (qd + 2) * qrows) * NBITS,
                                         qrows * NBITS)],
                        bufs[par], sems[par]).start()
            return 0

        lax.fori_loop(0, qch // 2, pair, 0)

        ocp = pltpu.make_async_copy(
            out_v, out_hbm.at[pl.ds(row0 * DIM, bpw * DIM)], sem_out)
        ocp.start()
        ocp.wait()

    return sc_kernel(w_flat, msg_flat)


def kernel(msg, emb_weight):
    n_batch, n_bits = msg.shape
    out = _sc_embed(emb_weight.reshape(-1), msg.reshape(-1), n_batch)
    return out.reshape(n_batch, DIM)


# speedup vs baseline: 1.0017x; 1.0008x over previous
import functools

import jax
import jax.numpy as jnp
from jax import lax
from jax.experimental import pallas as pl
from jax.experimental.pallas import tpu as pltpu
from jax.experimental.pallas import tpu_sc as plsc

NBITS = 100
DIM = 64
G = 6
NG = 17
TROWS = NG * 64 - 48
NC = 2
NS = 16
NW = NC * NS
LANES = 16
LASTROW = (NG - 1) * 64 * DIM


def _sc_embed(w_flat, msg_flat, n_batch):
    bpw = n_batch // NW
    qch = 8
    qrows = bpw // qch
    nbtq = qrows // LANES

    mesh = plsc.VectorSubcoreMesh(core_axis_name="c", subcore_axis_name="s")

    @functools.partial(
        pl.kernel,
        out_type=jax.ShapeDtypeStruct((n_batch * DIM,), jnp.float32),
        mesh=mesh,
        compiler_params=pltpu.CompilerParams(needs_layout_passes=False,
                                             use_tc_tiling_on_sc=True),
        scratch_types=[
            pltpu.VMEM((TROWS * DIM,), jnp.float32),
            pltpu.VMEM((NBITS * 2 * DIM,), jnp.float32),
            pltpu.VMEM((qrows * NBITS,), jnp.int32),
            pltpu.VMEM((qrows * NBITS,), jnp.int32),
            pltpu.VMEM((bpw * DIM,), jnp.float32),
            pltpu.VMEM((NG * LANES,), jnp.int32),
            pltpu.SemaphoreType.DMA,
            pltpu.SemaphoreType.DMA,
            pltpu.SemaphoreType.DMA,
            pltpu.SemaphoreType.DMA,
        ],
    )
    def sc_kernel(w_hbm, msg_hbm, out_hbm, t_v, w_v, m0_v, m1_v, out_v, mb_v,
                  sem_w, sem_m0, sem_m1, sem_out):
        cid = lax.axis_index("c")
        sid = lax.axis_index("s")
        wid = sid * NC + cid
        row0 = wid * bpw

        bufs = [m0_v, m1_v]
        sems = [sem_m0, sem_m1]

        wcp = pltpu.make_async_copy(w_hbm, w_v, sem_w)
        wcp.start()

        def msg_cp(q):
            return pltpu.make_async_copy(
                msg_hbm.at[pl.ds((row0 + q * qrows) * NBITS, qrows * NBITS)],
                bufs[q % 2], sems[q % 2])

        descs = {q: msg_cp(q) for q in range(2)}
        descs[0].start()
        descs[1].start()
        wcp.wait()

        zero = jnp.zeros((LANES,), jnp.float32)
        for s in range(DIM // LANES):
            base = lax.fori_loop(
                0, NBITS,
                lambda j, a, s=s: a + w_v[pl.ds(j * 2 * DIM + s * LANES, LANES)],
                zero)
            t_v[pl.ds(LASTROW + s * LANES, LANES)] = base

            def zrow(g, _, s=s):
                t_v[pl.ds(g * (64 * DIM) + s * LANES, LANES)] = zero
                return 0
            lax.fori_loop(0, NG - 1, zrow, 0)

        for k in range(G):
            ghi = NG if k < 4 else NG - 1

            def gstep(g, _, k=k):
                woff = (g * G + k) * 2 * DIM
                dsl = [w_v[pl.ds(woff + DIM + s * LANES, LANES)]
                       - w_v[pl.ds(woff + s * LANES, LANES)]
                       for s in range(DIM // LANES)]
                rb = g * (64 * DIM)

                def mstep(m, _):
                    src = rb + (m - (1 << k)) * DIM
                    dst = rb + m * DIM
                    for s in range(DIM // LANES):
                        t_v[pl.ds(dst + s * LANES, LANES)] = (
                            t_v[pl.ds(src + s * LANES, LANES)] + dsl[s])
                    return 0

                lax.fori_loop(1 << k, 2 << k, mstep, 0)
                return 0

            lax.fori_loop(0, ghi, gstep, 0)

        li = lax.iota(jnp.int32, LANES)

        def pair(p, _):
            for par in range(2):
                qd = 2 * p + par
                descs[par].wait()
                msg_v = bufs[par]

                def btile(bt, _, qd=qd, msg_v=msg_v):
                    ibase = (bt * LANES + li) * NBITS
                    obase = ((qd * qrows + bt * LANES) + li) * DIM
                    for g in range(NG):
                        nb = G if g < NG - 1 else NBITS - G * (NG - 1)
                        m = plsc.load_gather(msg_v, [ibase + G * g])
                        for i in range(1, nb):
                            bit = plsc.load_gather(msg_v, [ibase + (G * g + i)])
                            m = m + (bit << i)
                        mb_v[pl.ds(g * LANES, LANES)] = m

                    def cchunk(cc, _):
                        oadd = obase + cc * LANES
                        for g0, g1, first in ((0, 9, True), (9, NG, False)):
                            rbs = [
                                g * (64 * DIM) + cc * LANES
                                + mb_v[pl.ds(g * LANES, LANES)] * DIM
                                for g in range(g0, g1)
                            ]
                            for k in range(LANES):
                                sk = (li + k) & (LANES - 1)
                                acc = plsc.load_gather(t_v, [rbs[0] + sk])
                                for rb in rbs[1:]:
                                    acc = acc + plsc.load_gather(t_v, [rb + sk])
                                if first:
                                    plsc.store_scatter(out_v, [oadd + sk], acc)
                                else:
                                    plsc.addupdate_scatter(
                                        out_v, [oadd + sk], acc)
                        return 0

                    lax.fori_loop(0, DIM // LANES, cchunk, 0)
                    return 0

                lax.fori_loop(0, nbtq, btile, 0)

                @pl.when(qd + 2 < qch)
                def _start_next(qd=qd, par=par):
                    pltpu.make_async_copy(
                        msg_hbm.at[pl.ds((row0 + (qd + 2) * qrows) * NBITS,
                                         qrows * NBITS)],
                        bufs[par], sems[par]).start()
            return 0

        lax.fori_loop(0, qch // 2, pair, 0)

        ocp = pltpu.make_async_copy(
            out_v, out_hbm.at[pl.ds(row0 * DIM, bpw * DIM)], sem_out)
        ocp.start()
        ocp.wait()

    return sc_kernel(w_flat, msg_flat)


def kernel(msg, emb_weight):
    n_batch, n_bits = msg.shape
    out = _sc_embed(emb_weight.reshape(-1), msg.reshape(-1), n_batch)
    return out.reshape(n_batch, DIM)


# speedup vs baseline: 1.0372x; 1.0354x over previous
import functools

import jax
import jax.numpy as jnp
from jax import lax
from jax.experimental import pallas as pl
from jax.experimental.pallas import tpu as pltpu
from jax.experimental.pallas import tpu_sc as plsc

NBITS = 100
DIM = 64
G = 6
NG = 17
TROWS = NG * 64 - 48
NC = 2
NS = 16
NW = NC * NS
LANES = 16
LASTROW = (NG - 1) * 64 * DIM


def _sc_embed(w_flat, msg_flat, n_batch):
    bpw = n_batch // NW
    qch = 8
    qrows = bpw // qch
    nbtq = qrows // LANES

    mesh = plsc.VectorSubcoreMesh(core_axis_name="c", subcore_axis_name="s")

    @functools.partial(
        pl.kernel,
        out_type=jax.ShapeDtypeStruct((n_batch * DIM,), jnp.float32),
        mesh=mesh,
        compiler_params=pltpu.CompilerParams(needs_layout_passes=False),
        scratch_types=[
            pltpu.VMEM((TROWS * DIM,), jnp.float32),
            pltpu.VMEM((NBITS * 2 * DIM,), jnp.float32),
            pltpu.VMEM((qrows * NBITS,), jnp.int32),
            pltpu.VMEM((qrows * NBITS,), jnp.int32),
            pltpu.VMEM((bpw * DIM,), jnp.float32),
            pltpu.VMEM((NG * LANES,), jnp.int32),
            pltpu.SemaphoreType.DMA,
            pltpu.SemaphoreType.DMA,
            pltpu.SemaphoreType.DMA,
            pltpu.SemaphoreType.DMA,
        ],
    )
    def sc_kernel(w_hbm, msg_hbm, out_hbm, t_v, w_v, m0_v, m1_v, out_v, mb_v,
                  sem_w, sem_m0, sem_m1, sem_out):
        cid = lax.axis_index("c")
        sid = lax.axis_index("s")
        wid = sid * NC + cid
        row0 = wid * bpw

        bufs = [m0_v, m1_v]
        sems = [sem_m0, sem_m1]

        wcp = pltpu.make_async_copy(w_hbm, w_v, sem_w)
        wcp.start()

        def msg_cp(q):
            return pltpu.make_async_copy(
                msg_hbm.at[pl.ds((row0 + q * qrows) * NBITS, qrows * NBITS)],
                bufs[q % 2], sems[q % 2])

        descs = {q: msg_cp(q) for q in range(2)}
        descs[0].start()
        descs[1].start()
        wcp.wait()

        zero = jnp.zeros((LANES,), jnp.float32)
        for s in range(DIM // LANES):
            base = lax.fori_loop(
                0, NBITS,
                lambda j, a, s=s: a + w_v[pl.ds(j * 2 * DIM + s * LANES, LANES)],
                zero)
            t_v[pl.ds(LASTROW + s * LANES, LANES)] = base

            def zrow(g, _, s=s):
                t_v[pl.ds(g * (64 * DIM) + s * LANES, LANES)] = zero
                return 0
            lax.fori_loop(0, NG - 1, zrow, 0)

        for k in range(G):
            ghi = NG if k < 4 else NG - 1

            def gstep(g, _, k=k):
                woff = (g * G + k) * 2 * DIM
                dsl = [w_v[pl.ds(woff + DIM + s * LANES, LANES)]
                       - w_v[pl.ds(woff + s * LANES, LANES)]
                       for s in range(DIM // LANES)]
                rb = g * (64 * DIM)

                def mstep(m, _):
                    src = rb + (m - (1 << k)) * DIM
                    dst = rb + m * DIM
                    for s in range(DIM // LANES):
                        t_v[pl.ds(dst + s * LANES, LANES)] = (
                            t_v[pl.ds(src + s * LANES, LANES)] + dsl[s])
                    return 0

                lax.fori_loop(1 << k, 2 << k, mstep, 0)
                return 0

            lax.fori_loop(0, ghi, gstep, 0)

        def prow(r, _):
            for s in range(2):
                a = lax.bitcast_convert_type(
                    t_v[pl.ds(r * DIM + s * LANES, LANES)], jnp.int32)
                b = lax.bitcast_convert_type(
                    t_v[pl.ds(r * DIM + 32 + s * LANES, LANES)], jnp.int32)
                ar = (a + 0x7FFF + ((a >> 16) & 1)) & jnp.int32(-65536)
                br = lax.shift_right_logical(b + 0x7FFF + ((b >> 16) & 1), 16)
                packed = ar | br
                t_v[pl.ds(r * 32 + s * LANES, LANES)] = (
                    lax.bitcast_convert_type(packed, jnp.float32))
            return 0

        lax.fori_loop(0, TROWS, prow, 0)

        li = lax.iota(jnp.int32, LANES)
        himask = jnp.full((LANES,), jnp.int32(-65536))

        def pair(p, _):
            for par in range(2):
                qd = 2 * p + par
                descs[par].wait()
                msg_v = bufs[par]

                def btile(bt, _, qd=qd, msg_v=msg_v):
                    ibase = (bt * LANES + li) * NBITS
                    obase = ((qd * qrows + bt * LANES) + li) * DIM
                    for g in range(NG):
                        nb = G if g < NG - 1 else NBITS - G * (NG - 1)
                        m = plsc.load_gather(msg_v, [ibase + G * g])
                        for i in range(1, nb):
                            bit = plsc.load_gather(msg_v, [ibase + (G * g + i)])
                            m = m + (bit << i)
                        mb_v[pl.ds(g * LANES, LANES)] = m

                    def cchunk(cc, _):
                        oadd = obase + cc * LANES
                        for g0, g1, first in ((0, 9, True), (9, NG, False)):
                            rbs = [
                                g * (64 * 32) + cc * LANES
                                + mb_v[pl.ds(g * LANES, LANES)] * 32
                                for g in range(g0, g1)
                            ]
                            for k in range(LANES):
                                sk = (li + k) & (LANES - 1)
                                hi = None
                                lo = None
                                for rb in rbs:
                                    w = lax.bitcast_convert_type(
                                        plsc.load_gather(t_v, [rb + sk]),
                                        jnp.int32)
                                    h = lax.bitcast_convert_type(
                                        w & himask, jnp.float32)
                                    l2 = lax.bitcast_convert_type(
                                        lax.shift_left(w, 16), jnp.float32)
                                    hi = h if hi is None else hi + h
                                    lo = l2 if lo is None else lo + l2
                                if first:
                                    plsc.store_scatter(out_v, [oadd + sk], hi)
                                    plsc.store_scatter(
                                        out_v, [oadd + 32 + sk], lo)
                                else:
                                    plsc.addupdate_scatter(
                                        out_v, [oadd + sk], hi)
                                    plsc.addupdate_scatter(
                                        out_v, [oadd + 32 + sk], lo)
                        return 0

                    lax.fori_loop(0, 2, cchunk, 0)
                    return 0

                lax.fori_loop(0, nbtq, btile, 0)

                @pl.when(qd + 2 < qch)
                def _start_next(qd=qd, par=par):
                    pltpu.make_async_copy(
                        msg_hbm.at[pl.ds((row0 + (qd + 2) * qrows) * NBITS,
                                         qrows * NBITS)],
                        bufs[par], sems[par]).start()
            return 0

        lax.fori_loop(0, qch // 2, pair, 0)

        ocp = pltpu.make_async_copy(
            out_v, out_hbm.at[pl.ds(row0 * DIM, bpw * DIM)], sem_out)
        ocp.start()
        ocp.wait()

    return sc_kernel(w_flat, msg_flat)


def kernel(msg, emb_weight):
    n_batch, n_bits = msg.shape
    out = _sc_embed(emb_weight.reshape(-1), msg.reshape(-1), n_batch)
    return out.reshape(n_batch, DIM)


# speedup vs baseline: 1.4363x; 1.3848x over previous
import functools

import jax
import jax.numpy as jnp
from jax import lax
from jax.experimental import pallas as pl
from jax.experimental.pallas import tpu as pltpu
from jax.experimental.pallas import tpu_sc as plsc

NBITS = 100
DIM = 64
G = 6
NG = 17
TROWS = NG * 64 - 48
NC = 2
NS = 16
NW = NC * NS
LANES = 16


def _table_body(w_ref, t_ref):
    w = w_ref[...]
    diff = w[:, 1, :] - w[:, 0, :]
    basev = jnp.sum(w[:, 0, :], axis=0)
    r = lax.broadcasted_iota(jnp.int32, (TROWS, NBITS), 0)
    j = lax.broadcasted_iota(jnp.int32, (TROWS, NBITS), 1)
    grp = r // 64
    m = r % 64
    sel = (j // G == grp) & (((m >> (j % G)) & 1) == 1)
    mat = sel.astype(jnp.float32)
    t = lax.dot_general(mat, diff, (((1,), (0,)), ((), ())),
                        preferred_element_type=jnp.float32)
    is_last = (r[:, :1] >= (NG - 1) * 64).astype(jnp.float32)
    t = t + is_last * basev[None, :]
    ai = lax.bitcast_convert_type(t[:, :32], jnp.int32)
    bi = lax.bitcast_convert_type(t[:, 32:], jnp.int32)
    ar = (ai + 0x7FFF + ((ai >> 16) & 1)) & jnp.int32(-65536)
    br = lax.shift_right_logical(bi + 0x7FFF + ((bi >> 16) & 1), 16)
    t_ref[...] = ar | br


def _build_table(w3):
    return pl.pallas_call(
        _table_body,
        out_shape=jax.ShapeDtypeStruct((TROWS, 32), jnp.int32),
    )(w3)


def _sc_lookup(t_flat, msg_flat, n_batch):
    bpw = n_batch // NW
    qch = 4
    qrows = bpw // qch
    nbtq = qrows // LANES

    mesh = plsc.VectorSubcoreMesh(core_axis_name="c", subcore_axis_name="s")

    @functools.partial(
        pl.kernel,
        out_type=jax.ShapeDtypeStruct((n_batch * DIM,), jnp.float32),
        mesh=mesh,
        compiler_params=pltpu.CompilerParams(needs_layout_passes=False),
        scratch_types=[
            pltpu.VMEM((TROWS * 32,), jnp.int32),
            pltpu.VMEM((qrows * NBITS,), jnp.int32),
            pltpu.VMEM((qrows * NBITS,), jnp.int32),
            pltpu.VMEM((bpw * DIM,), jnp.float32),
            pltpu.VMEM((NG * LANES,), jnp.int32),
            pltpu.SemaphoreType.DMA,
            pltpu.SemaphoreType.DMA,
            pltpu.SemaphoreType.DMA,
            pltpu.SemaphoreType.DMA,
        ],
    )
    def sc_kernel(t_hbm, msg_hbm, out_hbm, t_v, m0_v, m1_v, out_v, mb_v,
                  sem_t, sem_m0, sem_m1, sem_out):
        cid = lax.axis_index("c")
        sid = lax.axis_index("s")
        wid = sid * NC + cid
        row0 = wid * bpw

        bufs = [m0_v, m1_v]
        sems = [sem_m0, sem_m1]

        tcp = pltpu.make_async_copy(t_hbm, t_v, sem_t)
        tcp.start()

        def msg_cp(q):
            return pltpu.make_async_copy(
                msg_hbm.at[pl.ds((row0 + q * qrows) * NBITS, qrows * NBITS)],
                bufs[q % 2], sems[q % 2])

        descs = {q: msg_cp(q) for q in range(qch)}
        descs[0].start()
        tcp.wait()
        li = lax.iota(jnp.int32, LANES)
        himask = jnp.full((LANES,), jnp.int32(-65536))

        for q in range(qch):
            descs[q].wait()
            if q + 1 < qch:
                descs[q + 1].start()
            msg_v = bufs[q % 2]

            def btile(bt, _, q=q, msg_v=msg_v):
                ibase = (bt * LANES + li) * NBITS
                obase = ((q * qrows + bt * LANES) + li) * DIM
                for g in range(NG):
                    nb = G if g < NG - 1 else NBITS - G * (NG - 1)
                    m = plsc.load_gather(msg_v, [ibase + G * g])
                    for i in range(1, nb):
                        bit = plsc.load_gather(msg_v, [ibase + (G * g + i)])
                        m = m + (bit << i)
                    mb_v[pl.ds(g * LANES, LANES)] = m

                def cchunk(cc, _):
                    oadd = obase + cc * LANES
                    rbs = [
                        g * (64 * 32) + cc * LANES
                        + mb_v[pl.ds(g * LANES, LANES)] * 32
                        for g in range(NG)
                    ]
                    for k in range(LANES):
                        sk = (li + k) & (LANES - 1)
                        hi = None
                        lo = None
                        for rb in rbs:
                            w = plsc.load_gather(t_v, [rb + sk])
                            h = lax.bitcast_convert_type(w & himask,
                                                         jnp.float32)
                            l2 = lax.bitcast_convert_type(
                                lax.shift_left(w, 16), jnp.float32)
                            hi = h if hi is None else hi + h
                            lo = l2 if lo is None else lo + l2
                        plsc.store_scatter(out_v, [oadd + sk], hi)
                        plsc.store_scatter(out_v, [oadd + 32 + sk], lo)
                    return 0

                lax.fori_loop(0, 2, cchunk, 0)
                return 0

            lax.fori_loop(0, nbtq, btile, 0)

        ocp = pltpu.make_async_copy(
            out_v, out_hbm.at[pl.ds(row0 * DIM, bpw * DIM)], sem_out)
        ocp.start()
        ocp.wait()

    return sc_kernel(t_flat, msg_flat)


def kernel(msg, emb_weight):
    n_batch, n_bits = msg.shape
    w3 = emb_weight.reshape(n_bits, 2, DIM)
    t = _build_table(w3)
    out = _sc_lookup(t.reshape(-1), msg.reshape(-1), n_batch)
    return out.reshape(n_batch, DIM)


# speedup vs baseline: 1.4500x; 1.0095x over previous
import functools

import jax
import jax.numpy as jnp
from jax import lax
from jax.experimental import pallas as pl
from jax.experimental.pallas import tpu as pltpu
from jax.experimental.pallas import tpu_sc as plsc

NBITS = 100
DIM = 64
G = 6
NG = 17
TROWS = NG * 64 - 48
NC = 2
NS = 16
NW = NC * NS
LANES = 16


def _table_body(w_ref, t_ref):
    w = w_ref[...]
    diff = w[:, 1, :] - w[:, 0, :]
    basev = jnp.sum(w[:, 0, :], axis=0)
    r = lax.broadcasted_iota(jnp.int32, (TROWS, NBITS), 0)
    j = lax.broadcasted_iota(jnp.int32, (TROWS, NBITS), 1)
    grp = r // 64
    m = r % 64
    sel = (j // G == grp) & (((m >> (j % G)) & 1) == 1)
    mat = sel.astype(jnp.float32)
    t = lax.dot_general(mat, diff, (((1,), (0,)), ((), ())),
                        preferred_element_type=jnp.float32)
    is_last = (r[:, :1] >= (NG - 1) * 64).astype(jnp.float32)
    t = t + is_last * basev[None, :]
    ai = lax.bitcast_convert_type(t[:, :32], jnp.int32)
    bi = lax.bitcast_convert_type(t[:, 32:], jnp.int32)
    ar = (ai + 0x7FFF + ((ai >> 16) & 1)) & jnp.int32(-65536)
    br = lax.shift_right_logical(bi + 0x7FFF + ((bi >> 16) & 1), 16)
    t_ref[...] = ar | br


def _build_table(w3):
    return pl.pallas_call(
        _table_body,
        out_shape=jax.ShapeDtypeStruct((TROWS, 32), jnp.int32),
    )(w3)


def _sc_lookup(t_flat, msg_flat, n_batch):
    bpw = n_batch // NW
    qch = 4
    qrows = bpw // qch
    nbtq = qrows // LANES

    mesh = plsc.VectorSubcoreMesh(core_axis_name="c", subcore_axis_name="s")

    @functools.partial(
        pl.kernel,
        out_type=jax.ShapeDtypeStruct((n_batch * DIM,), jnp.float32),
        mesh=mesh,
        compiler_params=pltpu.CompilerParams(needs_layout_passes=False),
        scratch_types=[
            pltpu.VMEM((TROWS * 32,), jnp.int32),
            pltpu.VMEM((qrows * NBITS,), jnp.int32),
            pltpu.VMEM((qrows * NBITS,), jnp.int32),
            pltpu.VMEM((bpw * DIM,), jnp.float32),
            pltpu.VMEM((NG * LANES,), jnp.int32),
            pltpu.SemaphoreType.DMA,
            pltpu.SemaphoreType.DMA,
            pltpu.SemaphoreType.DMA,
            pltpu.SemaphoreType.DMA,
        ],
    )
    def sc_kernel(t_hbm, msg_hbm, out_hbm, t_v, m0_v, m1_v, out_v, mb_v,
                  sem_t, sem_m0, sem_m1, sem_out):
        cid = lax.axis_index("c")
        sid = lax.axis_index("s")
        wid = sid * NC + cid
        row0 = wid * bpw

        bufs = [m0_v, m1_v]
        sems = [sem_m0, sem_m1]

        tcp = pltpu.make_async_copy(t_hbm, t_v, sem_t)
        tcp.start()

        def msg_cp(q):
            return pltpu.make_async_copy(
                msg_hbm.at[pl.ds((row0 + q * qrows) * NBITS, qrows * NBITS)],
                bufs[q % 2], sems[q % 2])

        descs = {q: msg_cp(q) for q in range(qch)}
        descs[0].start()
        tcp.wait()
        li = lax.iota(jnp.int32, LANES)
        himask = jnp.full((LANES,), jnp.int32(-65536))
        ocps = []

        for q in range(qch):
            descs[q].wait()
            if q + 1 < qch:
                descs[q + 1].start()
            msg_v = bufs[q % 2]

            def btile(bt, _, q=q, msg_v=msg_v):
                ibase = (bt * LANES + li) * NBITS
                obase = ((q * qrows + bt * LANES) + li) * DIM
                for g in range(NG):
                    nb = G if g < NG - 1 else NBITS - G * (NG - 1)
                    m = plsc.load_gather(msg_v, [ibase + G * g])
                    for i in range(1, nb):
                        bit = plsc.load_gather(msg_v, [ibase + (G * g + i)])
                        m = m + (bit << i)
                    mb_v[pl.ds(g * LANES, LANES)] = m

                def cchunk(cc, _):
                    oadd = obase + cc * LANES
                    rbs = [
                        g * (64 * 32) + cc * LANES
                        + mb_v[pl.ds(g * LANES, LANES)] * 32
                        for g in range(NG)
                    ]
                    for k in range(LANES):
                        sk = (li + k) & (LANES - 1)
                        hi = None
                        lo = None
                        for rb in rbs:
                            w = plsc.load_gather(t_v, [rb + sk])
                            h = lax.bitcast_convert_type(w & himask,
                                                         jnp.float32)
                            l2 = lax.bitcast_convert_type(
                                lax.shift_left(w, 16), jnp.float32)
                            hi = h if hi is None else hi + h
                            lo = l2 if lo is None else lo + l2
                        plsc.store_scatter(out_v, [oadd + sk], hi)
                        plsc.store_scatter(out_v, [oadd + 32 + sk], lo)
                    return 0

                lax.fori_loop(0, 2, cchunk, 0)
                return 0

            lax.fori_loop(0, nbtq, btile, 0)

            ocp = pltpu.make_async_copy(
                out_v.at[pl.ds(q * qrows * DIM, qrows * DIM)],
                out_hbm.at[pl.ds((row0 + q * qrows) * DIM, qrows * DIM)],
                sem_out)
            ocp.start()
            ocps.append(ocp)

        for ocp in ocps:
            ocp.wait()

    return sc_kernel(t_flat, msg_flat)


def kernel(msg, emb_weight):
    n_batch, n_bits = msg.shape
    w3 = emb_weight.reshape(n_bits, 2, DIM)
    t = _build_table(w3)
    out = _sc_lookup(t.reshape(-1), msg.reshape(-1), n_batch)
    return out.reshape(n_batch, DIM)


# speedup vs baseline: 1.7056x; 1.1763x over previous
import functools

import jax
import jax.numpy as jnp
from jax import lax
from jax.experimental import pallas as pl
from jax.experimental.pallas import tpu as pltpu
from jax.experimental.pallas import tpu_sc as plsc

NBITS = 100
DIM = 64
G = 6
NG = 17
TROWS = NG * 64 - 48
NC = 2
NS = 16
NW = NC * NS
LANES = 16


def _table_body(w_ref, t_ref):
    w = w_ref[...]
    diff = w[:, 1, :] - w[:, 0, :]
    basev = jnp.sum(w[:, 0, :], axis=0)
    r = lax.broadcasted_iota(jnp.int32, (TROWS, NBITS), 0)
    j = lax.broadcasted_iota(jnp.int32, (TROWS, NBITS), 1)
    grp = r // 64
    m = r % 64
    sel = (j // G == grp) & (((m >> (j % G)) & 1) == 1)
    mat = sel.astype(jnp.float32)
    t = lax.dot_general(mat, diff, (((1,), (0,)), ((), ())),
                        preferred_element_type=jnp.float32)
    is_last = (r[:, :1] >= (NG - 1) * 64).astype(jnp.float32)
    t = t + is_last * basev[None, :]
    ai = lax.bitcast_convert_type(t[:, :32], jnp.int32)
    bi = lax.bitcast_convert_type(t[:, 32:], jnp.int32)
    ar = (ai + 0x7FFF + ((ai >> 16) & 1)) & jnp.int32(-65536)
    br = lax.shift_right_logical(bi + 0x7FFF + ((bi >> 16) & 1), 16)
    t_ref[...] = ar | br


def _build_table(w3):
    return pl.pallas_call(
        _table_body,
        out_shape=jax.ShapeDtypeStruct((TROWS, 32), jnp.int32),
    )(w3)


def _sc_lookup(t_flat, msg_flat, n_batch):
    bpw = n_batch // NW
    qch = 4
    qrows = bpw // qch
    nbtq = qrows // LANES

    mesh = plsc.VectorSubcoreMesh(core_axis_name="c", subcore_axis_name="s")

    @functools.partial(
        pl.kernel,
        out_type=jax.ShapeDtypeStruct((n_batch * DIM,), jnp.float32),
        mesh=mesh,
        compiler_params=pltpu.CompilerParams(needs_layout_passes=False),
        scratch_types=[
            pltpu.VMEM((TROWS * 32,), jnp.int32),
            pltpu.VMEM((qrows * NBITS,), jnp.int32),
            pltpu.VMEM((qrows * NBITS,), jnp.int32),
            pltpu.VMEM((bpw * DIM,), jnp.float32),
            pltpu.VMEM((NG * LANES,), jnp.int32),
            pltpu.SemaphoreType.DMA,
            pltpu.SemaphoreType.DMA,
            pltpu.SemaphoreType.DMA,
            pltpu.SemaphoreType.DMA,
        ],
    )
    def sc_kernel(t_hbm, msg_hbm, out_hbm, t_v, m0_v, m1_v, out_v, mb_v,
                  sem_t, sem_m0, sem_m1, sem_out):
        cid = lax.axis_index("c")
        sid = lax.axis_index("s")
        wid = sid * NC + cid
        row0 = wid * bpw

        bufs = [m0_v, m1_v]
        sems = [sem_m0, sem_m1]

        tcp = pltpu.make_async_copy(t_hbm, t_v, sem_t)
        tcp.start()

        def msg_cp(q):
            return pltpu.make_async_copy(
                msg_hbm.at[pl.ds((row0 + q * qrows) * NBITS, qrows * NBITS)],
                bufs[q % 2], sems[q % 2])

        descs = {q: msg_cp(q) for q in range(qch)}
        descs[0].start()
        tcp.wait()
        li = lax.iota(jnp.int32, LANES)
        himask = jnp.full((LANES,), jnp.int32(-65536))
        ocps = []

        for q in range(qch):
            descs[q].wait()
            if q + 1 < qch:
                descs[q + 1].start()
            msg_v = bufs[q % 2]

            def btile(bt, _, q=q, msg_v=msg_v):
                ibase = (bt * LANES + li) * NBITS
                obase = ((q * qrows + bt * LANES) + li) * DIM
                for g in range(NG):
                    nb = G if g < NG - 1 else NBITS - G * (NG - 1)
                    m = plsc.load_gather(msg_v, [ibase + G * g])
                    for i in range(1, nb):
                        bit = plsc.load_gather(msg_v, [ibase + (G * g + i)])
                        m = m + (bit << i)
                    mb_v[pl.ds(g * LANES, LANES)] = m

                def cchunk(cc, _):
                    oadd = obase + cc * LANES
                    rbs = [
                        g * (64 * 32) + cc * LANES
                        + mb_v[pl.ds(g * LANES, LANES)] * 32
                        for g in range(NG)
                    ]
                    for k in range(LANES):
                        sk = (li + k) & (LANES - 1)
                        hi = None
                        lo = None
                        for rb in rbs:
                            w = plsc.load_gather(t_v, [rb + sk])
                            h = lax.bitcast_convert_type(w & himask,
                                                         jnp.float32)
                            l2 = lax.bitcast_convert_type(
                                lax.shift_left(w, 16), jnp.float32)
                            hi = h if hi is None else hi + h
                            lo = l2 if lo is None else lo + l2
                        plsc.store_scatter(out_v, [oadd + sk], hi)
                        plsc.store_scatter(out_v, [oadd + 32 + sk], lo)
                    return 0

                lax.fori_loop(0, 2, cchunk, 0)
                return 0

            lax.fori_loop(0, nbtq, btile, 0)

            ocp = pltpu.make_async_copy(
                out_v.at[pl.ds(q * qrows * DIM, qrows * DIM)],
                out_hbm.at[pl.ds((row0 + q * qrows) * DIM, qrows * DIM)],
                sem_out)
            ocp.start()
            ocps.append(ocp)

        for ocp in ocps:
            ocp.wait()

    return sc_kernel(t_flat, msg_flat)


def _tc_body(msg_ref, w_ref, out_ref):
    w = w_ref[...]
    diff = w[:, 1, :] - w[:, 0, :]
    base = jnp.sum(w[:, 0, :], axis=0)
    m = msg_ref[...].astype(jnp.float32)
    acc = lax.dot_general(m, diff, (((1,), (0,)), ((), ())),
                          preferred_element_type=jnp.float32)
    out_ref[...] = acc + base[None, :]


def _tc_matmul(msg_tc, w3):
    rows, n_bits = msg_tc.shape
    blk = 2048
    return pl.pallas_call(
        _tc_body,
        grid=(rows // blk,),
        in_specs=[
            pl.BlockSpec((blk, n_bits), lambda i: (i, 0)),
            pl.BlockSpec((n_bits, 2, DIM), lambda i: (0, 0, 0)),
        ],
        out_specs=pl.BlockSpec((blk, DIM), lambda i: (i, 0)),
        out_shape=jax.ShapeDtypeStruct((rows, DIM), jnp.float32),
    )(msg_tc, w3)


def kernel(msg, emb_weight):
    n_batch, n_bits = msg.shape
    w3 = emb_weight.reshape(n_bits, 2, DIM)
    n_sc = n_batch // 2
    t = _build_table(w3)
    out_sc = _sc_lookup(t.reshape(-1), msg[:n_sc].reshape(-1), n_sc)
    out_tc = _tc_matmul(msg[n_sc:], w3)
    return jnp.concatenate([out_sc.reshape(n_sc, DIM), out_tc], axis=0)


# speedup vs baseline: 2.2450x; 1.3162x over previous
import functools

import jax
import jax.numpy as jnp
from jax import lax
from jax.experimental import pallas as pl
from jax.experimental.pallas import tpu as pltpu
from jax.experimental.pallas import tpu_sc as plsc

NBITS = 100
DIM = 64
G = 6
NG = 17
TROWS = NG * 64 - 48
NC = 2
NS = 16
NW = NC * NS
LANES = 16


def _table_body(w_ref, t_ref):
    w = w_ref[...]
    diff = w[:, 1, :] - w[:, 0, :]
    basev = jnp.sum(w[:, 0, :], axis=0)
    r = lax.broadcasted_iota(jnp.int32, (TROWS, NBITS), 0)
    j = lax.broadcasted_iota(jnp.int32, (TROWS, NBITS), 1)
    grp = r // 64
    m = r % 64
    sel = (j // G == grp) & (((m >> (j % G)) & 1) == 1)
    mat = sel.astype(jnp.float32)
    t = lax.dot_general(mat, diff, (((1,), (0,)), ((), ())),
                        preferred_element_type=jnp.float32)
    is_last = (r[:, :1] >= (NG - 1) * 64).astype(jnp.float32)
    t = t + is_last * basev[None, :]
    ai = lax.bitcast_convert_type(t[:, :32], jnp.int32)
    bi = lax.bitcast_convert_type(t[:, 32:], jnp.int32)
    ar = (ai + 0x7FFF + ((ai >> 16) & 1)) & jnp.int32(-65536)
    br = lax.shift_right_logical(bi + 0x7FFF + ((bi >> 16) & 1), 16)
    t_ref[...] = ar | br


def _build_table(w3):
    return pl.pallas_call(
        _table_body,
        out_shape=jax.ShapeDtypeStruct((TROWS, 32), jnp.int32),
    )(w3)


def _sc_lookup(t_flat, msg_flat, n_batch):
    bpw = n_batch // NW
    qch = 4
    qrows = bpw // qch
    nbtq = qrows // LANES

    mesh = plsc.VectorSubcoreMesh(core_axis_name="c", subcore_axis_name="s")

    @functools.partial(
        pl.kernel,
        out_type=jax.ShapeDtypeStruct((n_batch * DIM,), jnp.float32),
        mesh=mesh,
        compiler_params=pltpu.CompilerParams(needs_layout_passes=False),
        scratch_types=[
            pltpu.VMEM((TROWS * 32,), jnp.int32),
            pltpu.VMEM((qrows * NBITS,), jnp.int32),
            pltpu.VMEM((qrows * NBITS,), jnp.int32),
            pltpu.VMEM((bpw * DIM,), jnp.float32),
            pltpu.VMEM((NG * LANES,), jnp.int32),
            pltpu.SemaphoreType.DMA,
            pltpu.SemaphoreType.DMA,
            pltpu.SemaphoreType.DMA,
            pltpu.SemaphoreType.DMA,
        ],
    )
    def sc_kernel(t_hbm, msg_hbm, out_hbm, t_v, m0_v, m1_v, out_v, mb_v,
                  sem_t, sem_m0, sem_m1, sem_out):
        cid = lax.axis_index("c")
        sid = lax.axis_index("s")
        wid = sid * NC + cid
        row0 = wid * bpw

        bufs = [m0_v, m1_v]
        sems = [sem_m0, sem_m1]

        tcp = pltpu.make_async_copy(t_hbm, t_v, sem_t)
        tcp.start()

        def msg_cp(q):
            return pltpu.make_async_copy(
                msg_hbm.at[pl.ds((row0 + q * qrows) * NBITS, qrows * NBITS)],
                bufs[q % 2], sems[q % 2])

        descs = {q: msg_cp(q) for q in range(qch)}
        descs[0].start()
        tcp.wait()
        li = lax.iota(jnp.int32, LANES)
        himask = jnp.full((LANES,), jnp.int32(-65536))
        ocps = []

        for q in range(qch):
            descs[q].wait()
            if q + 1 < qch:
                descs[q + 1].start()
            msg_v = bufs[q % 2]

            def btile(bt, _, q=q, msg_v=msg_v):
                ibase = (bt * LANES + li) * NBITS
                obase = ((q * qrows + bt * LANES) + li) * DIM
                for g in range(NG):
                    nb = G if g < NG - 1 else NBITS - G * (NG - 1)
                    m = plsc.load_gather(msg_v, [ibase + G * g])
                    for i in range(1, nb):
                        bit = plsc.load_gather(msg_v, [ibase + (G * g + i)])
                        m = m + (bit << i)
                    mb_v[pl.ds(g * LANES, LANES)] = m

                def cchunk(cc, _):
                    oadd = obase + cc * LANES
                    rbs = [
                        g * (64 * 32) + cc * LANES
                        + mb_v[pl.ds(g * LANES, LANES)] * 32
                        for g in range(NG)
                    ]
                    for k in range(LANES):
                        sk = (li + k) & (LANES - 1)
                        hi = None
                        lo = None
                        for rb in rbs:
                            w = plsc.load_gather(t_v, [rb + sk])
                            h = lax.bitcast_convert_type(w & himask,
                                                         jnp.float32)
                            l2 = lax.bitcast_convert_type(
                                lax.shift_left(w, 16), jnp.float32)
                            hi = h if hi is None else hi + h
                            lo = l2 if lo is None else lo + l2
                        plsc.store_scatter(out_v, [oadd + sk], hi)
                        plsc.store_scatter(out_v, [oadd + 32 + sk], lo)
                    return 0

                lax.fori_loop(0, 2, cchunk, 0)
                return 0

            lax.fori_loop(0, nbtq, btile, 0)

            ocp = pltpu.make_async_copy(
                out_v.at[pl.ds(q * qrows * DIM, qrows * DIM)],
                out_hbm.at[pl.ds((row0 + q * qrows) * DIM, qrows * DIM)],
                sem_out)
            ocp.start()
            ocps.append(ocp)

        for ocp in ocps:
            ocp.wait()

    return sc_kernel(t_flat, msg_flat)


def _tc_body(msg_ref, w_ref, out_ref):
    w = w_ref[...]
    diff = w[:, 1, :] - w[:, 0, :]
    base = jnp.sum(w[:, 0, :], axis=0)
    m = msg_ref[...].astype(jnp.float32)
    acc = lax.dot_general(m, diff, (((1,), (0,)), ((), ())),
                          preferred_element_type=jnp.float32)
    out_ref[...] = acc + base[None, :]


def _tc_matmul(msg_tc, w3):
    rows, n_bits = msg_tc.shape
    blk = 2048
    return pl.pallas_call(
        _tc_body,
        grid=(rows // blk,),
        in_specs=[
            pl.BlockSpec((blk, n_bits), lambda i: (i, 0)),
            pl.BlockSpec((n_bits, 2, DIM), lambda i: (0, 0, 0)),
        ],
        out_specs=pl.BlockSpec((blk, DIM), lambda i: (i, 0)),
        out_shape=jax.ShapeDtypeStruct((rows, DIM), jnp.float32),
    )(msg_tc, w3)


def kernel(msg, emb_weight):
    n_batch, n_bits = msg.shape
    w3 = emb_weight.reshape(n_bits, 2, DIM)
    n_sc = n_batch // 4
    t = _build_table(w3)
    out_sc = _sc_lookup(t.reshape(-1), msg[:n_sc].reshape(-1), n_sc)
    out_tc = _tc_matmul(msg[n_sc:], w3)
    return jnp.concatenate([out_sc.reshape(n_sc, DIM), out_tc], axis=0)
